# baseline (speedup 1.0000x reference)
import functools

import jax
import jax.numpy as jnp
from jax import lax
from jax.experimental import pallas as pl
from jax.experimental.pallas import tpu as pltpu
from jax.experimental.pallas import tpu_sc as plsc

N = 10000
D = 128
E = 320000
NC = 2
NS = 16
NP = 10240
NQ = 2560
ARPT = NQ // NS
KCH = 128
CHUNKS = 2560
EPAD = CHUNKS * KCH
PGRP = CHUNKS // NS
SLOTS = 4
IGN = 1 << 30


@functools.cache
def _sc_mesh():
    return plsc.VectorSubcoreMesh(
        core_axis_name="c", subcore_axis_name="s", num_cores=NC,
        num_subcores=NS,
    )


@functools.cache
def _prop_kernel():
    return pl.kernel(
        _prop_body,
        out_type=jax.ShapeDtypeStruct((NC, NQ, D), jnp.float32),
        mesh=_sc_mesh(),
        scratch_types=[
            pltpu.VMEM((PGRP, KCH), jnp.int32),
            pltpu.VMEM((PGRP, KCH), jnp.int32),
            pltpu.VMEM((SLOTS, KCH, D), jnp.float32),
            pltpu.VMEM((16, D), jnp.float32),
            pltpu.VMEM_SHARED((NQ, D), jnp.float32),
            pltpu.SemaphoreType.DMA,
        ],
    )


def _prop_body(tab_hbm, src_hbm, dst_hbm, out_hbm, src_v, dst_v, rows_v,
               zb_v, acc_sp, sem):
    c = lax.axis_index("c")
    s = lax.axis_index("s")
    for i in range(16):
        for l in range(D // 16):
            zb_v[i, pl.ds(l * 16, 16)] = jnp.zeros((16,), jnp.float32)

    def zero_step(k, carry):
        pltpu.sync_copy(zb_v, acc_sp.at[pl.ds(s * ARPT + k * 16, 16)])
        return carry

    lax.fori_loop(0, ARPT // 16, zero_step, 0)
    pltpu.sync_copy(src_hbm.at[c, pl.ds(s * PGRP, PGRP)], src_v)
    pltpu.sync_copy(dst_hbm.at[c, pl.ds(s * PGRP, PGRP)], dst_v)
    plsc.subcore_barrier()

    def group_step(g, carry):
        base = g * SLOTS
        descs = [
            pltpu.async_copy(
                tab_hbm.at[plsc.Indices(src_v.at[base + b],
                                        ignored_value=IGN)],
                rows_v.at[b], sem)
            for b in range(SLOTS)
        ]
        for d in descs:
            d.wait()
        for b in range(SLOTS):
            pltpu.sync_copy(
                rows_v.at[b],
                acc_sp.at[plsc.Indices(dst_v.at[base + b],
                                       ignored_value=IGN)],
                add=True)
        return carry

    lax.fori_loop(0, PGRP // SLOTS, group_step, 0)
    plsc.subcore_barrier()
    pltpu.sync_copy(
        acc_sp.at[pl.ds(s * ARPT, ARPT)], out_hbm.at[c, pl.ds(s * ARPT, ARPT)]
    )


def _dis_from_quarters(acca, accb):
    def body(acca_ref, accb_ref, dis_ref):
        deg = jnp.concatenate(
            [acca_ref[0, 0:NQ, 0:1], acca_ref[1, 0:NQ, 0:1],
             accb_ref[0, 0:NQ, 0:1], accb_ref[1, 0:N - 3 * NQ, 0:1]], axis=0
        ) + 1.0
        dis_ref[...] = lax.rsqrt(deg)

    return pl.pallas_call(
        body, out_shape=jax.ShapeDtypeStruct((N, 1), jnp.float32)
    )(acca, accb)


def _dense_pre(h, w, dis):
    def body(h_ref, w_ref, dis_ref, tab_ref):
        tmp = jnp.dot(h_ref[...], w_ref[...],
                      preferred_element_type=jnp.float32)
        tab_ref[0:N, :] = tmp * dis_ref[...]
        tab_ref[N:NP, :] = jnp.zeros((NP - N, D), jnp.float32)

    return pl.pallas_call(
        body, out_shape=jax.ShapeDtypeStruct((NP, D), jnp.float32)
    )(h, w, dis)


def _dense_post(acca, accb, tab, dis, b, g, be):
    def body(acca_ref, accb_ref, tab_ref, dis_ref, b_ref, g_ref, be_ref,
             h_ref):
        dis = dis_ref[...]
        accfull = jnp.concatenate(
            [acca_ref[0, 0:NQ, :], acca_ref[1, 0:NQ, :],
             accb_ref[0, 0:NQ, :], accb_ref[1, 0:N - 3 * NQ, :]], axis=0
        )
        pre = dis * (accfull + tab_ref[0:N, :]) + b_ref[...]
        mu = jnp.mean(pre, axis=0, keepdims=True)
        xc = pre - mu
        var = jnp.mean(xc * xc, axis=0, keepdims=True)
        bn = g_ref[...] * (xc / jnp.sqrt(var + 1e-5)) + be_ref[...]
        h_ref[...] = jnp.where(bn > 0, bn, jnp.exp(bn) - 1.0)

    return pl.pallas_call(
        body, out_shape=jax.ShapeDtypeStruct((N, D), jnp.float32)
    )(acca, accb, tab, dis, b, g, be)


def kernel(x, edge_index, W1, b1, g1, be1, W2, b2, g2, be2):
    src = edge_index[0].astype(jnp.int32)
    dst = edge_index[1].astype(jnp.int32)
    pad = jnp.full((EPAD - E,), N, jnp.int32)
    src1 = jnp.concatenate([src, pad])
    dst1 = jnp.concatenate([dst, pad])

    def masked(q, deg):
        base = q * NQ
        inq = (dst1 >= base) & (dst1 < base + NQ)
        if deg:
            inq = inq & (dst1 < N)
            s = jnp.where(inq, 0, IGN)
        else:
            s = jnp.where(inq, src1, IGN)
        d = jnp.where(inq, dst1 - base, IGN)
        return (s.reshape(CHUNKS, KCH), d.reshape(CHUNKS, KCH))

    sd = [[masked(2 * h + c, dg) for c in range(NC)]
          for dg in (True, False) for h in range(2)]
    src_da, src_db = (jnp.stack([sd[i][0][0], sd[i][1][0]]) for i in (0, 1))
    dst_da, dst_db = (jnp.stack([sd[i][0][1], sd[i][1][1]]) for i in (0, 1))
    src_pa, src_pb = (jnp.stack([sd[i][0][0], sd[i][1][0]]) for i in (2, 3))
    dst_pa, dst_pb = (jnp.stack([sd[i][0][1], sd[i][1][1]]) for i in (2, 3))

    src_x = jnp.stack([src_da, src_db, src_pa, src_pb, src_pa, src_pb])
    dst_x = jnp.stack([dst_da, dst_db, dst_pa, dst_pb, dst_pa, dst_pb])

    ws = jnp.stack([W1, W1, W1, W1, W2, W2])
    bs = jnp.stack([b1, b1, b1, b1, b2, b2]).reshape(6, 1, D)
    gs = jnp.stack([g1, g1, g1, g1, g2, g2]).reshape(6, 1, D)
    bes = jnp.stack([be1, be1, be1, be1, be2, be2]).reshape(6, 1, D)
    f_ones = jnp.asarray([1, 1, 0, 0, 0, 0], jnp.float32)
    f_fresh = jnp.asarray([0, 0, 1, 0, 1, 0], jnp.float32)
    f_dis = jnp.asarray([0, 1, 0, 0, 0, 0], jnp.float32)
    f_hout = jnp.asarray([0, 0, 0, 1, 0, 1], jnp.float32)

    ones_tab = jnp.ones((NP, D), jnp.float32)

    def step(carry, xi):
        h, tab, acca, dis = carry
        w, b, g, be, srci, dsti, fo, ff, fd, fh = xi
        tab_new = _dense_pre(h, w, dis)
        tab_use = jnp.where(fo > 0, ones_tab,
                            jnp.where(ff > 0, tab_new, tab))
        acc = _prop_kernel()(tab_use, srci, dsti)
        dis_new = _dis_from_quarters(acca, acc)
        dis_next = jnp.where(fd > 0, dis_new, dis)
        h_new = _dense_post(acca, acc, tab_use, dis_next, b, g, be)
        h_next = jnp.where(fh > 0, h_new, h)
        return (h_next, tab_use, acc, dis_next), h_next

    zt = jnp.zeros((NP, D), jnp.float32)
    za = jnp.zeros((NC, NQ, D), jnp.float32)
    dis0 = jnp.ones((N, 1), jnp.float32)
    _, hs = lax.scan(
        step, (x, zt, za, dis0),
        (ws, bs, gs, bes, src_x, dst_x, f_ones, f_fresh, f_dis, f_hout))
    return (x, hs[3], hs[5])


# speedup vs baseline: 1.7525x; 1.7525x over previous
import functools

import jax
import jax.numpy as jnp
from jax import lax
from jax.experimental import pallas as pl
from jax.experimental.pallas import tpu as pltpu
from jax.experimental.pallas import tpu_sc as plsc

N = 10000
D = 128
E = 320000
NC = 2
NS = 16
NP = 10240
NQ = 2560
ARPT = NQ // NS
KCH = 128
CHUNKS = 2560
EPAD = CHUNKS * KCH
PGRP = CHUNKS // NS
SLOTS = 4
NGRP = PGRP // SLOTS
IGN = 1 << 30


@functools.cache
def _sc_mesh():
    return plsc.VectorSubcoreMesh(
        core_axis_name="c", subcore_axis_name="s", num_cores=NC,
        num_subcores=NS,
    )


@functools.cache
def _prop_kernel():
    return pl.kernel(
        _prop_body,
        out_type=jax.ShapeDtypeStruct((NC, NQ, D), jnp.float32),
        mesh=_sc_mesh(),
        scratch_types=[
            pltpu.VMEM((PGRP, KCH), jnp.int32),
            pltpu.VMEM((PGRP, KCH), jnp.int32),
            pltpu.VMEM((SLOTS, KCH, D), jnp.float32),
            pltpu.VMEM((16, D), jnp.float32),
            pltpu.VMEM_SHARED((NQ, D), jnp.float32),
            pltpu.SemaphoreType.DMA,
        ],
    )


def _prop_body(tab_hbm, src_hbm, dst_hbm, out_hbm, src_v, dst_v, rows_v,
               zb_v, acc_sp, sem):
    c = lax.axis_index("c")
    s = lax.axis_index("s")
    for i in range(16):
        for l in range(D // 16):
            zb_v[i, pl.ds(l * 16, 16)] = jnp.zeros((16,), jnp.float32)

    def zero_step(k, carry):
        pltpu.sync_copy(zb_v, acc_sp.at[pl.ds(s * ARPT + k * 16, 16)])
        return carry

    lax.fori_loop(0, ARPT // 16, zero_step, 0)
    pltpu.sync_copy(src_hbm.at[c, pl.ds(s * PGRP, PGRP)], src_v)
    pltpu.sync_copy(dst_hbm.at[c, pl.ds(s * PGRP, PGRP)], dst_v)
    plsc.subcore_barrier()

    def group_step(g, carry):
        base = g * SLOTS
        gats = [
            pltpu.async_copy(
                tab_hbm.at[plsc.Indices(src_v.at[base + b],
                                        ignored_value=IGN)],
                rows_v.at[b], sem)
            for b in range(SLOTS)
        ]
        for d in gats:
            d.wait()
        for b in range(SLOTS):
            pltpu.sync_copy(
                rows_v.at[b],
                acc_sp.at[plsc.Indices(dst_v.at[base + b],
                                       ignored_value=IGN)],
                add=True)
        return carry

    lax.fori_loop(0, NGRP, group_step, 0)
    plsc.subcore_barrier()
    pltpu.sync_copy(
        acc_sp.at[pl.ds(s * ARPT, ARPT)], out_hbm.at[c, pl.ds(s * ARPT, ARPT)]
    )


def _dis_from_onehot(acc):
    def body(acc_ref, dis_ref):
        cols = [
            acc_ref[0, :, 32 * q:32 * q + 1] + acc_ref[1, :, 32 * q:32 * q + 1]
            for q in range(4)
        ]
        deg = jnp.concatenate(cols, axis=0)[0:N] + 1.0
        dis_ref[...] = lax.rsqrt(deg)

    return pl.pallas_call(
        body, out_shape=jax.ShapeDtypeStruct((N, 1), jnp.float32)
    )(acc)


def _dense_pre(h, w, dis):
    def body(h_ref, w_ref, dis_ref, tab_ref):
        tmp = jnp.dot(h_ref[...], w_ref[...],
                      preferred_element_type=jnp.float32)
        tab_ref[0:N, :] = tmp * dis_ref[...]
        tab_ref[N:NP, :] = jnp.zeros((NP - N, D), jnp.float32)

    return pl.pallas_call(
        body, out_shape=jax.ShapeDtypeStruct((NP, D), jnp.float32)
    )(h, w, dis)


def _dense_post(acca, accb, tab, dis, b, g, be):
    def body(acca_ref, accb_ref, tab_ref, dis_ref, b_ref, g_ref, be_ref,
             h_ref):
        dis = dis_ref[...]
        accfull = jnp.concatenate(
            [acca_ref[0, 0:NQ, :], acca_ref[1, 0:NQ, :],
             accb_ref[0, 0:NQ, :], accb_ref[1, 0:N - 3 * NQ, :]], axis=0
        )
        pre = dis * (accfull + tab_ref[0:N, :]) + b_ref[...]
        mu = jnp.mean(pre, axis=0, keepdims=True)
        xc = pre - mu
        var = jnp.mean(xc * xc, axis=0, keepdims=True)
        bn = g_ref[...] * (xc / jnp.sqrt(var + 1e-5)) + be_ref[...]
        h_ref[...] = jnp.where(bn > 0, bn, jnp.exp(bn) - 1.0)

    return pl.pallas_call(
        body, out_shape=jax.ShapeDtypeStruct((N, D), jnp.float32)
    )(acca, accb, tab, dis, b, g, be)


def kernel(x, edge_index, W1, b1, g1, be1, W2, b2, g2, be2):
    src = edge_index[0].astype(jnp.int32)
    dst = edge_index[1].astype(jnp.int32)
    pad = jnp.full((EPAD - E,), N, jnp.int32)
    src1 = jnp.concatenate([src, pad])
    dst1 = jnp.concatenate([dst, pad])

    def maskedq(q):
        base = q * NQ
        inq = (dst1 >= base) & (dst1 < base + NQ)
        s = jnp.where(inq, src1, IGN)
        d = jnp.where(inq, dst1 - base, IGN)
        return (s.reshape(CHUNKS, KCH), d.reshape(CHUNKS, KCH))

    q0, q1, q2, q3 = (maskedq(q) for q in range(4))
    src_pa = jnp.stack([q0[0], q1[0]])
    dst_pa = jnp.stack([q0[1], q1[1]])
    src_pb = jnp.stack([q2[0], q3[0]])
    dst_pb = jnp.stack([q2[1], q3[1]])

    pos = jnp.arange(EPAD, dtype=jnp.int32)
    halves = [(pos < EPAD // 2), (pos >= EPAD // 2)]
    sdeg, ddeg = [], []
    for c in range(NC):
        valid = halves[c] & (dst1 < N)
        sdeg.append(jnp.where(valid, dst1 // NQ, IGN).reshape(CHUNKS, KCH))
        ddeg.append(jnp.where(valid, dst1 % NQ, IGN).reshape(CHUNKS, KCH))
    src_dg = jnp.stack(sdeg)
    dst_dg = jnp.stack(ddeg)

    src_x = jnp.stack([src_dg, src_dg, src_pa, src_pb, src_pa, src_pb])
    dst_x = jnp.stack([dst_dg, dst_dg, dst_pa, dst_pb, dst_pa, dst_pb])

    ws = jnp.stack([W1, W1, W1, W1, W2, W2])
    bs = jnp.stack([b1, b1, b1, b1, b2, b2]).reshape(6, 1, D)
    gs = jnp.stack([g1, g1, g1, g1, g2, g2]).reshape(6, 1, D)
    bes = jnp.stack([be1, be1, be1, be1, be2, be2]).reshape(6, 1, D)
    f_ones = jnp.asarray([1, 1, 0, 0, 0, 0], jnp.float32)
    f_fresh = jnp.asarray([0, 0, 1, 0, 1, 0], jnp.float32)
    f_dis = jnp.asarray([0, 1, 0, 0, 0, 0], jnp.float32)
    f_hout = jnp.asarray([0, 0, 0, 1, 0, 1], jnp.float32)

    oh = jnp.zeros((NP, D), jnp.float32)
    for q in range(4):
        oh = oh.at[q, 32 * q:32 * (q + 1)].set(1.0)

    def step(carry, xi):
        h, tab, acca, dis = carry
        w, b, g, be, srci, dsti, fo, ff, fd, fh = xi
        tab_new = _dense_pre(h, w, dis)
        tab_use = jnp.where(fo > 0, oh, jnp.where(ff > 0, tab_new, tab))
        acc = _prop_kernel()(tab_use, srci, dsti)
        dis_new = _dis_from_onehot(acc)
        dis_next = jnp.where(fd > 0, dis_new, dis)
        h_new = _dense_post(acca, acc, tab_use, dis_next, b, g, be)
        h_next = jnp.where(fh > 0, h_new, h)
        return (h_next, tab_use, acc, dis_next), h_next

    zt = jnp.zeros((NP, D), jnp.float32)
    za = jnp.zeros((NC, NQ, D), jnp.float32)
    dis0 = jnp.ones((N, 1), jnp.float32)
    _, hs = lax.scan(
        step, (x, zt, za, dis0),
        (ws, bs, gs, bes, src_x, dst_x, f_ones, f_fresh, f_dis, f_hout))
    return (x, hs[3], hs[5])


# speedup vs baseline: 2.7023x; 1.5420x over previous
import functools

import jax
import jax.numpy as jnp
from jax import lax
from jax.experimental import pallas as pl
from jax.experimental.pallas import tpu as pltpu
from jax.experimental.pallas import tpu_sc as plsc

N = 10000
D = 128
E = 320000
NC = 2
NS = 16
NP = 10240
NQ = 2560
ARPT = NQ // NS
KCH = 128
CHUNKS = 2560
EPAD = CHUNKS * KCH
PGRP = CHUNKS // NS
SLOTS = 4
NGRP = PGRP // SLOTS
IGN = 1 << 30


@functools.cache
def _sc_mesh():
    return plsc.VectorSubcoreMesh(
        core_axis_name="c", subcore_axis_name="s", num_cores=NC,
        num_subcores=NS,
    )


@functools.cache
def _prop_kernel():
    return pl.kernel(
        _prop_body,
        out_type=jax.ShapeDtypeStruct((NC, NQ, D), jnp.float32),
        mesh=_sc_mesh(),
        scratch_types=[
            pltpu.VMEM((PGRP, KCH), jnp.int32),
            pltpu.VMEM((PGRP, KCH), jnp.int32),
            pltpu.VMEM((SLOTS, KCH, D), jnp.float32),
            pltpu.VMEM((16, D), jnp.float32),
            pltpu.VMEM_SHARED((NQ, D), jnp.float32),
            pltpu.SemaphoreType.DMA,
        ],
    )


def _prop_body(tab_hbm, src_hbm, dst_hbm, out_hbm, src_v, dst_v, rows_v,
               zb_v, acc_sp, sem):
    c = lax.axis_index("c")
    s = lax.axis_index("s")
    for i in range(16):
        for l in range(D // 16):
            zb_v[i, pl.ds(l * 16, 16)] = jnp.zeros((16,), jnp.float32)

    def zero_step(k, carry):
        pltpu.sync_copy(zb_v, acc_sp.at[pl.ds(s * ARPT + k * 16, 16)])
        return carry

    lax.fori_loop(0, ARPT // 16, zero_step, 0)
    pltpu.sync_copy(src_hbm.at[c, pl.ds(s * PGRP, PGRP)], src_v)
    pltpu.sync_copy(dst_hbm.at[c, pl.ds(s * PGRP, PGRP)], dst_v)
    plsc.subcore_barrier()

    def group_step(g, carry):
        base = g * SLOTS
        gats = [
            pltpu.async_copy(
                tab_hbm.at[plsc.Indices(src_v.at[base + b],
                                        ignored_value=IGN)],
                rows_v.at[b], sem)
            for b in range(SLOTS)
        ]
        for d in gats:
            d.wait()
        for b in range(SLOTS):
            pltpu.sync_copy(
                rows_v.at[b],
                acc_sp.at[plsc.Indices(dst_v.at[base + b],
                                       ignored_value=IGN)],
                add=True)
        return carry

    lax.fori_loop(0, NGRP, group_step, 0)
    plsc.subcore_barrier()
    pltpu.sync_copy(
        acc_sp.at[pl.ds(s * ARPT, ARPT)], out_hbm.at[c, pl.ds(s * ARPT, ARPT)]
    )


def _dis_from_onehot(acc):
    def body(acc_ref, dis_ref):
        cols = [
            acc_ref[0, :, 32 * q:32 * q + 1] + acc_ref[1, :, 32 * q:32 * q + 1]
            for q in range(4)
        ]
        deg = jnp.concatenate(cols, axis=0)[0:N] + 1.0
        dis_ref[...] = lax.rsqrt(deg)

    return pl.pallas_call(
        body, out_shape=jax.ShapeDtypeStruct((N, 1), jnp.float32)
    )(acc)


def _dense_pre(h, w, dis):
    def body(h_ref, w_ref, dis_ref, tab_ref):
        tmp = jnp.dot(h_ref[...], w_ref[...],
                      preferred_element_type=jnp.float32)
        tab_ref[0:N, :] = tmp * dis_ref[...]
        tab_ref[N:NP, :] = jnp.zeros((NP - N, D), jnp.float32)

    return pl.pallas_call(
        body, out_shape=jax.ShapeDtypeStruct((NP, D), jnp.float32)
    )(h, w, dis)


def _dense_post(acca, accb, tab, dis, b, g, be):
    def body(acca_ref, accb_ref, tab_ref, dis_ref, b_ref, g_ref, be_ref,
             h_ref):
        dis = dis_ref[...]
        accfull = jnp.concatenate(
            [acca_ref[0, 0:NQ, :], acca_ref[1, 0:NQ, :],
             accb_ref[0, 0:NQ, :], accb_ref[1, 0:N - 3 * NQ, :]], axis=0
        )
        pre = dis * (accfull + tab_ref[0:N, :]) + b_ref[...]
        mu = jnp.mean(pre, axis=0, keepdims=True)
        xc = pre - mu
        var = jnp.mean(xc * xc, axis=0, keepdims=True)
        bn = g_ref[...] * (xc / jnp.sqrt(var + 1e-5)) + be_ref[...]
        h_ref[...] = jnp.where(bn > 0, bn, jnp.exp(bn) - 1.0)

    return pl.pallas_call(
        body, out_shape=jax.ShapeDtypeStruct((N, D), jnp.float32)
    )(acca, accb, tab, dis, b, g, be)


def kernel(x, edge_index, W1, b1, g1, be1, W2, b2, g2, be2):
    src = edge_index[0].astype(jnp.int32)
    dst = edge_index[1].astype(jnp.int32)
    pad = jnp.full((EPAD - E,), N, jnp.int32)
    src1 = jnp.concatenate([src, pad])
    dst1 = jnp.concatenate([dst, pad])

    def maskedq(q):
        base = q * NQ
        inq = (dst1 >= base) & (dst1 < base + NQ)
        s = jnp.where(inq, src1, IGN)
        d = jnp.where(inq, dst1 - base, IGN)
        return (s.reshape(CHUNKS, KCH), d.reshape(CHUNKS, KCH))

    q0, q1, q2, q3 = (maskedq(q) for q in range(4))
    src_pa = jnp.stack([q0[0], q1[0]])
    dst_pa = jnp.stack([q0[1], q1[1]])
    src_pb = jnp.stack([q2[0], q3[0]])
    dst_pb = jnp.stack([q2[1], q3[1]])

    pos = jnp.arange(EPAD, dtype=jnp.int32)
    halves = [(pos < EPAD // 2), (pos >= EPAD // 2)]
    sdeg, ddeg = [], []
    for c in range(NC):
        valid = halves[c] & (dst1 < N)
        sdeg.append(jnp.where(valid, dst1 // NQ, IGN).reshape(CHUNKS, KCH))
        ddeg.append(jnp.where(valid, dst1 % NQ, IGN).reshape(CHUNKS, KCH))
    src_dg = jnp.stack(sdeg)
    dst_dg = jnp.stack(ddeg)

    src_x = jnp.stack([src_dg, src_pa, src_pb, src_pa, src_pb])
    dst_x = jnp.stack([dst_dg, dst_pa, dst_pb, dst_pa, dst_pb])

    ws = jnp.stack([W1, W1, W1, W2, W2])
    bs = jnp.stack([b1, b1, b1, b2, b2]).reshape(5, 1, D)
    gs = jnp.stack([g1, g1, g1, g2, g2]).reshape(5, 1, D)
    bes = jnp.stack([be1, be1, be1, be2, be2]).reshape(5, 1, D)
    f_ones = jnp.asarray([1, 0, 0, 0, 0], jnp.float32)
    f_fresh = jnp.asarray([0, 1, 0, 1, 0], jnp.float32)
    f_dis = jnp.asarray([1, 0, 0, 0, 0], jnp.float32)
    f_hout = jnp.asarray([0, 0, 1, 0, 1], jnp.float32)

    oh = jnp.zeros((NP, D), jnp.float32)
    for q in range(4):
        oh = oh.at[q, 32 * q:32 * (q + 1)].set(1.0)

    def step(carry, xi):
        h, tab, acca, dis = carry
        w, b, g, be, srci, dsti, fo, ff, fd, fh = xi
        tab_new = _dense_pre(h, w, dis)
        tab_use = jnp.where(fo > 0, oh, jnp.where(ff > 0, tab_new, tab))
        acc = _prop_kernel()(tab_use, srci, dsti)
        dis_new = _dis_from_onehot(acc)
        dis_next = jnp.where(fd > 0, dis_new, dis)
        h_new = _dense_post(acca, acc, tab_use, dis_next, b, g, be)
        h_next = jnp.where(fh > 0, h_new, h)
        return (h_next, tab_use, acc, dis_next), h_next

    zt = jnp.zeros((NP, D), jnp.float32)
    za = jnp.zeros((NC, NQ, D), jnp.float32)
    dis0 = jnp.ones((N, 1), jnp.float32)
    _, hs = lax.scan(
        step, (x, zt, za, dis0),
        (ws, bs, gs, bes, src_x, dst_x, f_ones, f_fresh, f_dis, f_hout))
    return (x, hs[2], hs[4])
